# Initial kernel scaffold; baseline (speedup 1.0000x reference)
import jax
import jax.numpy as jnp
from jax.experimental import pallas as pl


def kernel(X_A, X_B, idxA, wA, idxB, wB, E, W_A2B, b_A2B, W_B2A, b_B2A, Wg_A, bg_A, Wg_B, bg_B):
    raise NotImplementedError("write your pallas kernel here")


# speedup vs baseline: 16.4276x; 16.4276x over previous
import functools
import math

import jax
import jax.numpy as jnp
from jax.experimental import pallas as pl
from jax.experimental.pallas import tpu as pltpu

_B, _N, _D, _E, _KE = 2, 2048, 1024, 16, 8
_NT = 512


def _assign_tile(idx, w, nt):
    iota_e = jax.lax.broadcasted_iota(jnp.int32, (nt, _E), 1)
    acc = jnp.zeros((nt, _E), jnp.float32)
    for k in range(_KE):
        acc = acc + jnp.where(idx[:, k:k + 1] == iota_e, w[:, k:k + 1], 0.0)
    return acc


def _gelu_exact(x):
    return 0.5 * x * (1.0 + jax.lax.erf(x * (1.0 / math.sqrt(2.0))))


def _he_body(idxA_ref, wA_ref, idxB_ref, wB_ref, xA_ref, xB_ref,
             heA_ref, heB_ref):
    nt = pl.program_id(1)
    A_t = _assign_tile(idxA_ref[0], wA_ref[0], _NT)
    B_t = _assign_tile(idxB_ref[0], wB_ref[0], _NT)
    dn = (((0,), (0,)), ((), ()))
    heA = jax.lax.dot_general(A_t, xA_ref[0], dn,
                              preferred_element_type=jnp.float32)
    heB = jax.lax.dot_general(B_t, xB_ref[0], dn,
                              preferred_element_type=jnp.float32)

    @pl.when(nt == 0)
    def _():
        heA_ref[0] = heA
        heB_ref[0] = heB

    @pl.when(nt != 0)
    def _():
        heA_ref[0] += heA
        heB_ref[0] += heB


def _edge_body(heA_ref, heB_ref, wb2a_ref, bb2a_ref, wa2b_ref, ba2b_ref,
               wgA_bot_ref, wgB_bot_ref, heAt_ref, heBt_ref, mA_ref, mB_ref):
    heAt = _gelu_exact(jnp.dot(heA_ref[...], wb2a_ref[...],
                               preferred_element_type=jnp.float32)
                       + bb2a_ref[...])
    heBt = _gelu_exact(jnp.dot(heB_ref[...], wa2b_ref[...],
                               preferred_element_type=jnp.float32)
                       + ba2b_ref[...])
    heAt_ref[...] = heAt
    heBt_ref[...] = heBt
    mA_ref[...] = jnp.dot(heBt, wgA_bot_ref[...],
                          preferred_element_type=jnp.float32)
    mB_ref[...] = jnp.dot(heAt, wgB_bot_ref[...],
                          preferred_element_type=jnp.float32)


def _main_body(idxA_ref, wA_ref, idxB_ref, wB_ref, xA_ref, xB_ref,
               wgA_top_ref, wgB_top_ref, bgA_ref, bgB_ref,
               heAt_ref, heBt_ref, mA_ref, mB_ref,
               outA_ref, outB_ref):
    A_t = _assign_tile(idxA_ref[0], wA_ref[0], _NT)
    B_t = _assign_tile(idxB_ref[0], wB_ref[0], _NT)
    xA = xA_ref[0]
    xB = xB_ref[0]
    preA = (jnp.dot(xA, wgA_top_ref[...], preferred_element_type=jnp.float32)
            + jnp.dot(A_t, mA_ref[...], preferred_element_type=jnp.float32)
            + bgA_ref[...])
    gA = jax.nn.sigmoid(preA)
    xAfromB = jnp.dot(A_t, heBt_ref[...], preferred_element_type=jnp.float32)
    outA_ref[0] = gA * xA + (1.0 - gA) * xAfromB

    preB = (jnp.dot(xB, wgB_top_ref[...], preferred_element_type=jnp.float32)
            + jnp.dot(B_t, mB_ref[...], preferred_element_type=jnp.float32)
            + bgB_ref[...])
    gB = jax.nn.sigmoid(preB)
    xBfromA = jnp.dot(B_t, heAt_ref[...], preferred_element_type=jnp.float32)
    outB_ref[0] = gB * xB + (1.0 - gB) * xBfromA


def kernel(X_A, X_B, idxA, wA, idxB, wB, E, W_A2B, b_A2B, W_B2A, b_B2A,
           Wg_A, bg_A, Wg_B, bg_B):
    del E
    f32 = jnp.float32
    nnt = _N // _NT

    idx_spec = pl.BlockSpec((1, _NT, _KE), lambda b, n: (b, n, 0))
    x_spec = pl.BlockSpec((1, _NT, _D), lambda b, n: (b, n, 0))
    he_spec = pl.BlockSpec((1, _E, _D), lambda b, n: (b, 0, 0))

    he_A, he_B = pl.pallas_call(
        _he_body,
        grid=(_B, nnt),
        in_specs=[idx_spec, idx_spec, idx_spec, idx_spec, x_spec, x_spec],
        out_specs=[he_spec, he_spec],
        out_shape=[jax.ShapeDtypeStruct((_B, _E, _D), f32),
                   jax.ShapeDtypeStruct((_B, _E, _D), f32)],
    )(idxA, wA, idxB, wB, X_A, X_B)

    he_A2 = he_A.reshape(_B * _E, _D)
    he_B2 = he_B.reshape(_B * _E, _D)
    full = lambda shp: pl.BlockSpec(shp, lambda: tuple(0 for _ in shp))
    he_At, he_Bt, m_A, m_B = pl.pallas_call(
        _edge_body,
        grid=(),
        in_specs=[full((_B * _E, _D)), full((_B * _E, _D)),
                  full((_D, _D)), full((1, _D)),
                  full((_D, _D)), full((1, _D)),
                  full((_D, _D)), full((_D, _D))],
        out_specs=[full((_B * _E, _D))] * 4,
        out_shape=[jax.ShapeDtypeStruct((_B * _E, _D), f32)] * 4,
    )(he_A2, he_B2, W_B2A, b_B2A.reshape(1, _D), W_A2B, b_A2B.reshape(1, _D),
      Wg_A[_D:], Wg_B[_D:])

    wg_top_spec = pl.BlockSpec((_D, _D), lambda b, n: (0, 0))
    bias_spec = pl.BlockSpec((1, _D), lambda b, n: (0, 0))
    edge_spec = pl.BlockSpec((_E, _D), lambda b, n: (b, 0))
    out_A, out_B = pl.pallas_call(
        _main_body,
        grid=(_B, nnt),
        in_specs=[idx_spec, idx_spec, idx_spec, idx_spec, x_spec, x_spec,
                  wg_top_spec, wg_top_spec, bias_spec, bias_spec,
                  edge_spec, edge_spec, edge_spec, edge_spec],
        out_specs=[x_spec, x_spec],
        out_shape=[jax.ShapeDtypeStruct((_B, _N, _D), f32),
                   jax.ShapeDtypeStruct((_B, _N, _D), f32)],
    )(idxA, wA, idxB, wB, X_A, X_B,
      Wg_A[:_D], Wg_B[:_D], bg_A.reshape(1, _D), bg_B.reshape(1, _D),
      he_At, he_Bt, m_A, m_B)

    return (out_A, out_B)


# speedup vs baseline: 31.4035x; 1.9116x over previous
import math

import jax
import jax.numpy as jnp
from jax.experimental import pallas as pl
from jax.experimental.pallas import tpu as pltpu

_B, _N, _D, _E, _KE = 2, 2048, 1024, 16, 8
_NT = 512

_DN0 = (((0,), (0,)), ((), ()))


def _assign_tile_t(idxT, wT):
    nt = idxT.shape[-1]
    iota_e = jax.lax.broadcasted_iota(jnp.int32, (_E, nt), 0)
    acc = jnp.zeros((_E, nt), jnp.float32)
    for k in range(_KE):
        acc = acc + jnp.where(idxT[k:k + 1, :] == iota_e, wT[k:k + 1, :], 0.0)
    return acc


def _gelu_exact(x):
    return 0.5 * x * (1.0 + jax.lax.erf(x * (1.0 / math.sqrt(2.0))))


def _he_body(idxAT_ref, wAT_ref, idxBT_ref, wBT_ref, xA_ref, xB_ref,
             heA_ref, heB_ref):
    nt = pl.program_id(1)
    At = _assign_tile_t(idxAT_ref[0], wAT_ref[0])
    Bt = _assign_tile_t(idxBT_ref[0], wBT_ref[0])
    heA = jnp.dot(At, xA_ref[0], preferred_element_type=jnp.float32)
    heB = jnp.dot(Bt, xB_ref[0], preferred_element_type=jnp.float32)

    @pl.when(nt == 0)
    def _():
        heA_ref[0] = heA
        heB_ref[0] = heB

    @pl.when(nt != 0)
    def _():
        heA_ref[0] += heA
        heB_ref[0] += heB


def _main_body(idxAT_ref, wAT_ref, idxBT_ref, wBT_ref, xA_ref, xB_ref,
               heA_ref, heB_ref, wb2a_ref, bb2a_ref, wa2b_ref, ba2b_ref,
               wgA_top_ref, wgA_bot_ref, wgB_top_ref, wgB_bot_ref,
               bgA_ref, bgB_ref,
               outA_ref, outB_ref,
               heAt_s, heBt_s, mA_s, mB_s):
    @pl.when(pl.program_id(1) == 0)
    def _():
        heAt = _gelu_exact(
            jnp.dot(heA_ref[0], wb2a_ref[...],
                    preferred_element_type=jnp.float32) + bb2a_ref[...])
        heBt = _gelu_exact(
            jnp.dot(heB_ref[0], wa2b_ref[...],
                    preferred_element_type=jnp.float32) + ba2b_ref[...])
        heAt_s[...] = heAt
        heBt_s[...] = heBt
        mA_s[...] = jnp.dot(heBt, wgA_bot_ref[...],
                            preferred_element_type=jnp.float32)
        mB_s[...] = jnp.dot(heAt, wgB_bot_ref[...],
                            preferred_element_type=jnp.float32)

    At = _assign_tile_t(idxAT_ref[0], wAT_ref[0])
    Bt = _assign_tile_t(idxBT_ref[0], wBT_ref[0])

    xA = xA_ref[0]
    preA = (jnp.dot(xA, wgA_top_ref[...], preferred_element_type=jnp.float32)
            + jax.lax.dot_general(At, mA_s[...], _DN0,
                                  preferred_element_type=jnp.float32)
            + bgA_ref[...])
    gA = jax.nn.sigmoid(preA)
    xAfromB = jax.lax.dot_general(At, heBt_s[...], _DN0,
                                  preferred_element_type=jnp.float32)
    outA_ref[0] = gA * xA + (1.0 - gA) * xAfromB

    xB = xB_ref[0]
    preB = (jnp.dot(xB, wgB_top_ref[...], preferred_element_type=jnp.float32)
            + jax.lax.dot_general(Bt, mB_s[...], _DN0,
                                  preferred_element_type=jnp.float32)
            + bgB_ref[...])
    gB = jax.nn.sigmoid(preB)
    xBfromA = jax.lax.dot_general(Bt, heAt_s[...], _DN0,
                                  preferred_element_type=jnp.float32)
    outB_ref[0] = gB * xB + (1.0 - gB) * xBfromA


def kernel(X_A, X_B, idxA, wA, idxB, wB, E, W_A2B, b_A2B, W_B2A, b_B2A,
           Wg_A, bg_A, Wg_B, bg_B):
    del E
    f32 = jnp.float32
    nnt = _N // _NT

    idxAT = jnp.swapaxes(idxA, 1, 2)
    wAT = jnp.swapaxes(wA, 1, 2)
    idxBT = jnp.swapaxes(idxB, 1, 2)
    wBT = jnp.swapaxes(wB, 1, 2)

    idxt_spec = pl.BlockSpec((1, _KE, _NT), lambda b, n: (b, 0, n))
    x_spec = pl.BlockSpec((1, _NT, _D), lambda b, n: (b, n, 0))
    he_spec = pl.BlockSpec((1, _E, _D), lambda b, n: (b, 0, 0))

    he_A, he_B = pl.pallas_call(
        _he_body,
        grid=(_B, nnt),
        in_specs=[idxt_spec, idxt_spec, idxt_spec, idxt_spec, x_spec, x_spec],
        out_specs=[he_spec, he_spec],
        out_shape=[jax.ShapeDtypeStruct((_B, _E, _D), f32),
                   jax.ShapeDtypeStruct((_B, _E, _D), f32)],
    )(idxAT, wAT, idxBT, wBT, X_A, X_B)

    w_spec = pl.BlockSpec((_D, _D), lambda b, n: (0, 0))
    wg_top_spec = pl.BlockSpec((_D, _D), lambda b, n: (0, 0))
    wg_bot_spec = pl.BlockSpec((_D, _D), lambda b, n: (1, 0))
    bias_spec = pl.BlockSpec((1, _D), lambda b, n: (0, 0))
    scr = pltpu.VMEM((_E, _D), f32)

    out_A, out_B = pl.pallas_call(
        _main_body,
        grid=(_B, nnt),
        in_specs=[idxt_spec, idxt_spec, idxt_spec, idxt_spec, x_spec, x_spec,
                  he_spec, he_spec,
                  w_spec, bias_spec, w_spec, bias_spec,
                  wg_top_spec, wg_bot_spec, wg_top_spec, wg_bot_spec,
                  bias_spec, bias_spec],
        out_specs=[x_spec, x_spec],
        out_shape=[jax.ShapeDtypeStruct((_B, _N, _D), f32),
                   jax.ShapeDtypeStruct((_B, _N, _D), f32)],
        scratch_shapes=[scr, scr, scr, scr],
    )(idxAT, wAT, idxBT, wBT, X_A, X_B, he_A, he_B,
      W_B2A, b_B2A.reshape(1, _D), W_A2B, b_A2B.reshape(1, _D),
      Wg_A, Wg_A, Wg_B, Wg_B,
      bg_A.reshape(1, _D), bg_B.reshape(1, _D))

    return (out_A, out_B)
